# Initial kernel scaffold; baseline (speedup 1.0000x reference)
import functools

import jax
import jax.numpy as jnp
from jax import lax
from jax.experimental import pallas as pl
from jax.experimental.pallas import tpu as pltpu
from jax.experimental.pallas import tpu_sc as plsc

NC = 2
NS = 16
CHUNK = 128
LANES = 16


def _make_sc_agg(n_pad, d, e_pad, with_cnt):
    nchunks = e_pad // CHUNK
    nloops = -(-nchunks // (NC * NS))
    rows_per_tile = n_pad // NS

    out_type = [jax.ShapeDtypeStruct((NC, n_pad, d), jnp.float32)]
    if with_cnt:
        out_type.append(jax.ShapeDtypeStruct((NC, n_pad), jnp.float32))
    scratch = [
        pltpu.VMEM((CHUNK,), jnp.int32),
        pltpu.VMEM((CHUNK,), jnp.int32),
        pltpu.VMEM((CHUNK, d), jnp.float32),
        pltpu.VMEM((CHUNK,), jnp.float32),
        pltpu.VMEM_SHARED((n_pad, d), jnp.float32),
        pltpu.VMEM_SHARED((n_pad,), jnp.float32),
        pltpu.SemaphoreType.DMA,
    ]
    mesh = plsc.VectorSubcoreMesh(core_axis_name="c", subcore_axis_name="s")

    @functools.partial(pl.kernel, out_type=out_type, mesh=mesh,
                       scratch_types=scratch)
    def k(x_hbm, src_hbm, dst_hbm, zrows_hbm, zcnt_hbm, out_agg, *rest):
        if with_cnt:
            out_cnt, sidx, didx, rows, ones, acc, cacc, sem = rest
        else:
            sidx, didx, rows, ones, acc, cacc, sem = rest
        c = lax.axis_index("c")
        s = lax.axis_index("s")
        wid = c * NS + s
        r0 = s * rows_per_tile
        pltpu.sync_copy(zrows_hbm.at[pl.ds(r0, rows_per_tile)],
                        acc.at[pl.ds(r0, rows_per_tile)])
        if with_cnt:
            pltpu.sync_copy(zcnt_hbm.at[pl.ds(r0, rows_per_tile)],
                            cacc.at[pl.ds(r0, rows_per_tile)])
            for i in range(CHUNK // LANES):
                ones[pl.ds(i * LANES, LANES)] = jnp.ones((LANES,), jnp.float32)
        plsc.subcore_barrier()

        def body(i, carry):
            ci = wid + i * (NC * NS)

            @pl.when(ci < nchunks)
            def _():
                base = ci * CHUNK
                pltpu.sync_copy(src_hbm.at[pl.ds(base, CHUNK)], sidx)
                pltpu.sync_copy(dst_hbm.at[pl.ds(base, CHUNK)], didx)
                pltpu.async_copy(x_hbm.at[sidx], rows, sem).wait()
                pltpu.sync_copy(rows, acc.at[didx], add=True)
                if with_cnt:
                    pltpu.sync_copy(ones, cacc.at[didx], add=True)
            return carry

        lax.fori_loop(0, nloops, body, 0)
        plsc.subcore_barrier()
        pltpu.sync_copy(acc.at[pl.ds(r0, rows_per_tile)],
                        out_agg.at[c, pl.ds(r0, rows_per_tile)])
        if with_cnt:
            pltpu.sync_copy(cacc.at[pl.ds(r0, rows_per_tile)],
                            out_cnt.at[c, pl.ds(r0, rows_per_tile)])

    return k


def _tc_layer(final, agg0, agg1, cnt0, cnt1, xin, wl_t, wr_t, b):
    n_pad, d = xin.shape
    blk = 1024

    def body(a0, a1, c0, c1, x, wl, wr, bb, out):
        cnt = jnp.maximum(c0[...] + c1[...], 1.0)
        mean = (a0[...] + a1[...]) / cnt
        z = (jnp.dot(mean, wl[...], preferred_element_type=jnp.float32)
             + bb[...]
             + jnp.dot(x[...], wr[...], preferred_element_type=jnp.float32))
        if final:
            m = jnp.max(z, axis=-1, keepdims=True)
            ez = jnp.exp(z - m)
            out[...] = (z - m) - jnp.log(jnp.sum(ez, axis=-1, keepdims=True))
        else:
            out[...] = jnp.maximum(z, 0.0)

    row_spec = pl.BlockSpec((blk, d), lambda i: (i, 0))
    col_spec = pl.BlockSpec((blk, 1), lambda i: (i, 0))
    full_spec = pl.BlockSpec((d, d), lambda i: (0, 0))
    b_spec = pl.BlockSpec((1, d), lambda i: (0, 0))
    return pl.pallas_call(
        body,
        grid=(n_pad // blk,),
        in_specs=[row_spec, row_spec, col_spec, col_spec, row_spec,
                  full_spec, full_spec, b_spec],
        out_specs=row_spec,
        out_shape=jax.ShapeDtypeStruct((n_pad, d), jnp.float32),
    )(agg0, agg1, cnt0, cnt1, xin, wl_t, wr_t, b)


def kernel(x, edge_index, W1l, b1, W1r, W2l, b2, W2r):
    n, d = x.shape
    e = edge_index.shape[1]
    n_pad = -(-(n + 1) // 2048) * 2048
    e_pad = -(-e // CHUNK) * CHUNK

    src = edge_index[0].astype(jnp.int32)
    dst = edge_index[1].astype(jnp.int32)
    if e_pad != e:
        src = jnp.concatenate([src, jnp.zeros((e_pad - e,), jnp.int32)])
        dst = jnp.concatenate(
            [dst, jnp.full((e_pad - e,), n_pad - 1, jnp.int32)])

    zrows = jnp.zeros((n_pad, d), jnp.float32)
    zcnt = jnp.zeros((n_pad,), jnp.float32)
    x_pad = jnp.concatenate([x, jnp.zeros((n_pad - n, d), jnp.float32)])

    sc_agg_cnt = _make_sc_agg(n_pad, d, e_pad, True)
    sc_agg = _make_sc_agg(n_pad, d, e_pad, False)

    aggp, cntp = sc_agg_cnt(x_pad, src, dst, zrows, zcnt)
    cnt0 = cntp[0].reshape(n_pad, 1)
    cnt1 = cntp[1].reshape(n_pad, 1)
    h = _tc_layer(False, aggp[0], aggp[1], cnt0, cnt1, x_pad,
                  W1l.T, W1r.T, b1.reshape(1, d))
    aggp2 = sc_agg(h, src, dst, zrows, zcnt)
    out = _tc_layer(True, aggp2[0], aggp2[1], cnt0, cnt1, h,
                    W2l.T, W2r.T, b2.reshape(1, d))
    return out[:n]


# speedup vs baseline: 6.5304x; 6.5304x over previous
import functools

import jax
import jax.numpy as jnp
from jax import lax
from jax.experimental import pallas as pl
from jax.experimental.pallas import tpu as pltpu
from jax.experimental.pallas import tpu_sc as plsc

NC = 2
NS = 16
CHUNK = 128
LANES = 16


def _make_sc_agg(n_pad, d, e_pad, with_cnt):
    nchunks = e_pad // CHUNK
    nloops = -(-nchunks // (NC * NS))
    rows_per_tile = n_pad // NS

    out_type = [jax.ShapeDtypeStruct((NC, n_pad, d), jnp.float32)]
    if with_cnt:
        out_type.append(jax.ShapeDtypeStruct((NC, n_pad), jnp.float32))
    scratch = [
        pltpu.VMEM((CHUNK,), jnp.int32),
        pltpu.VMEM((CHUNK,), jnp.int32),
        pltpu.VMEM((CHUNK, d), jnp.float32),
        pltpu.VMEM((CHUNK,), jnp.float32),
        pltpu.VMEM_SHARED((n_pad, d), jnp.float32),
        pltpu.VMEM_SHARED((n_pad,), jnp.float32),
        pltpu.SemaphoreType.DMA,
    ]
    mesh = plsc.VectorSubcoreMesh(core_axis_name="c", subcore_axis_name="s")

    @functools.partial(pl.kernel, out_type=out_type, mesh=mesh,
                       scratch_types=scratch)
    def k(x_hbm, src_hbm, dst_hbm, zrows_hbm, zcnt_hbm, out_agg, *rest):
        if with_cnt:
            out_cnt, sidx, didx, rows, ones, acc, cacc, sem = rest
        else:
            sidx, didx, rows, ones, acc, cacc, sem = rest
        c = lax.axis_index("c")
        s = lax.axis_index("s")
        wid = c * NS + s
        r0 = s * rows_per_tile
        pltpu.sync_copy(zrows_hbm.at[pl.ds(r0, rows_per_tile)],
                        acc.at[pl.ds(r0, rows_per_tile)])
        if with_cnt:
            pltpu.sync_copy(zcnt_hbm.at[pl.ds(r0, rows_per_tile)],
                            cacc.at[pl.ds(r0, rows_per_tile)])
            for i in range(CHUNK // LANES):
                ones[pl.ds(i * LANES, LANES)] = jnp.ones((LANES,), jnp.float32)
        plsc.subcore_barrier()

        def body(i, carry):
            ci = wid + i * (NC * NS)

            @pl.when(ci < nchunks)
            def _():
                base = ci * CHUNK
                pltpu.sync_copy(src_hbm.at[pl.ds(base, CHUNK)], sidx)
                pltpu.sync_copy(dst_hbm.at[pl.ds(base, CHUNK)], didx)
                pltpu.async_copy(x_hbm.at[sidx], rows, sem).wait()
                pltpu.sync_copy(rows, acc.at[didx], add=True)
                if with_cnt:
                    pltpu.sync_copy(ones, cacc.at[didx], add=True)
            return carry

        lax.fori_loop(0, nloops, body, 0)
        plsc.subcore_barrier()
        pltpu.sync_copy(acc.at[pl.ds(r0, rows_per_tile)],
                        out_agg.at[c, pl.ds(r0, rows_per_tile)])
        if with_cnt:
            pltpu.sync_copy(cacc.at[pl.ds(r0, rows_per_tile)],
                            out_cnt.at[c, pl.ds(r0, rows_per_tile)])

    return k


def _tc_layer(final, agg0, agg1, cnt0, cnt1, xin, wl_t, wr_t, b):
    n_pad, d = xin.shape
    blk = 1024

    def body(a0, a1, c0, c1, x, wl, wr, bb, out):
        cnt = jnp.maximum(c0[...] + c1[...], 1.0)
        mean = (a0[...] + a1[...]) / cnt
        z = (jnp.dot(mean, wl[...], preferred_element_type=jnp.float32)
             + bb[...]
             + jnp.dot(x[...], wr[...], preferred_element_type=jnp.float32))
        if final:
            m = jnp.max(z, axis=-1, keepdims=True)
            ez = jnp.exp(z - m)
            out[...] = (z - m) - jnp.log(jnp.sum(ez, axis=-1, keepdims=True))
        else:
            out[...] = jnp.maximum(z, 0.0)

    row_spec = pl.BlockSpec((blk, d), lambda i: (i, 0))
    col_spec = pl.BlockSpec((blk, 1), lambda i: (i, 0))
    full_spec = pl.BlockSpec((d, d), lambda i: (0, 0))
    b_spec = pl.BlockSpec((1, d), lambda i: (0, 0))
    return pl.pallas_call(
        body,
        grid=(n_pad // blk,),
        in_specs=[row_spec, row_spec, col_spec, col_spec, row_spec,
                  full_spec, full_spec, b_spec],
        out_specs=row_spec,
        out_shape=jax.ShapeDtypeStruct((n_pad, d), jnp.float32),
    )(agg0, agg1, cnt0, cnt1, xin, wl_t, wr_t, b)


def kernel(x, edge_index, W1l, b1, W1r, W2l, b2, W2r):
    n, d = x.shape
    e = edge_index.shape[1]
    n_pad = -(-(n + 1) // 2048) * 2048
    e_pad = -(-e // CHUNK) * CHUNK

    src = edge_index[0].astype(jnp.int32)
    dst = edge_index[1].astype(jnp.int32)
    if e_pad != e:
        src = jnp.concatenate([src, jnp.zeros((e_pad - e,), jnp.int32)])
        dst = jnp.concatenate(
            [dst, jnp.full((e_pad - e,), n_pad - 1, jnp.int32)])

    zrows = jnp.zeros((n_pad, d), jnp.float32)
    zcnt = jnp.zeros((n_pad,), jnp.float32)
    x_pad = jnp.concatenate([x, jnp.zeros((n_pad - n, d), jnp.float32)])

    sc_agg_cnt = _make_sc_agg(n_pad, d, e_pad, True)
    sc_agg = _make_sc_agg(n_pad, d, e_pad, False)

    aggp, cntp = sc_agg_cnt(x_pad, src, dst, zrows, zcnt)
    cnt0 = cntp[0].reshape(n_pad, 1)
    cnt1 = cntp[1].reshape(n_pad, 1)
    h = _tc_layer(False, aggp[0], aggp[1], cnt0, cnt1, x_pad,
                  W1l.T, W1r.T, b1.reshape(1, d))
    (aggp2,) = sc_agg(h, src, dst, zrows, zcnt)
    out = _tc_layer(True, aggp2[0], aggp2[1], cnt0, cnt1, h,
                    W2l.T, W2r.T, b2.reshape(1, d))
    return out[:n]
